# baseline (speedup 1.0000x reference)
import jax
import jax.numpy as jnp
from jax import lax
from jax.experimental import pallas as pl
from jax.experimental.pallas import tpu as pltpu
from jax.experimental.pallas import tpu_sc as plsc

N_NODES = 10000
N_EDGES = 320000
D_IN = 128
D_OUT = 128

NC = 2
NS = 16
NW = NC * NS
EPW = 10240
PAD_E = NW * EPW - N_EDGES
CH = 80
NCH = EPW // CH
CHD = 64
NCHD = EPW // CHD
NBUF_D = 5
N_PAD = 10240
RPT = N_PAD // NS
NWO = RPT // CH
NWOD = RPT // CHD
DEG_W = 128


def _acc_body(feat_hbm, src_hbm, dst_hbm, zfeat_hbm,
              acc_out,
              idx_s, idx_d, rows, sem, acc_sh):
    c = lax.axis_index("c")
    s = lax.axis_index("s")
    wid = s * NC + c
    base = wid * EPW
    row0 = s * RPT

    pltpu.sync_copy(zfeat_hbm, rows)

    def zinit(k, carry):
        pltpu.sync_copy(rows, acc_sh.at[pl.ds(row0 + k * CH, CH)])
        return carry

    lax.fori_loop(0, NWO, zinit, 0)
    plsc.subcore_barrier()

    def chunk(j, carry):
        off = base + j * CH
        pltpu.sync_copy(src_hbm.at[pl.ds(off, CH)], idx_s)
        pltpu.sync_copy(dst_hbm.at[pl.ds(off, CH)], idx_d)
        pltpu.async_copy(feat_hbm.at[idx_s], rows, sem).wait()
        pltpu.sync_copy(rows, acc_sh.at[idx_d], add=True)
        return carry

    lax.fori_loop(0, NCH, chunk, 0)
    plsc.subcore_barrier()

    def wout(k, carry):
        r = row0 + k * CH
        pltpu.sync_copy(acc_sh.at[pl.ds(r, CH)], rows)
        pltpu.sync_copy(rows, acc_out.at[c, pl.ds(r, CH)])
        return carry

    lax.fori_loop(0, NWO, wout, 0)


def _deg_body(dst_hbm, zdeg_hbm, ones_hbm,
              deg_out,
              idx_d, ones_v, wbuf, semid, semsc, semw, deg_sh):
    c = lax.axis_index("c")
    s = lax.axis_index("s")
    wid = s * NC + c
    base = wid * EPW
    row0 = s * RPT

    def idx_start(j, b):
        pltpu.async_copy(dst_hbm.at[pl.ds(base + j * CHD, CHD)], idx_d[b], semid[b])

    def idx_wait(j, b):
        pltpu.make_async_copy(
            dst_hbm.at[pl.ds(base + j * CHD, CHD)], idx_d[b], semid[b]
        ).wait()

    def scatter_start(b):
        pltpu.async_copy(ones_v, deg_sh.at[idx_d[b]], semsc[b], add=True)

    def scatter_wait(b):
        pltpu.make_async_copy(ones_v, deg_sh.at[idx_d[b]], semsc[b]).wait()

    pltpu.sync_copy(ones_hbm, ones_v)
    pltpu.sync_copy(zdeg_hbm, wbuf[0])

    def zinit(k, carry):
        pltpu.sync_copy(wbuf[0], deg_sh.at[pl.ds(row0 + k * CHD, CHD)])
        return carry

    lax.fori_loop(0, NWOD, zinit, 0)
    plsc.subcore_barrier()

    idx_start(0, 0)
    idx_start(1, 1)

    def grp(g, carry):
        for b in range(NBUF_D):
            j = g * NBUF_D + b
            bi = (b + 2) % NBUF_D

            @pl.when(j >= 3)
            def _():
                scatter_wait(bi)

            @pl.when(j + 2 < NCHD)
            def _():
                idx_start(j + 2, bi)

            idx_wait(j, b)
            scatter_start(b)
        return carry

    lax.fori_loop(0, NCHD // NBUF_D, grp, 0)
    for j0 in range(NCHD - 3, NCHD):
        scatter_wait(j0 % NBUF_D)
    plsc.subcore_barrier()

    for k in range(NWOD):
        b = k % 2
        if k >= 2:
            pltpu.make_async_copy(
                wbuf[b], deg_out.at[c, pl.ds(row0 + (k - 2) * CHD, CHD)], semw[b]
            ).wait()
        r = row0 + k * CHD
        pltpu.async_copy(deg_sh.at[pl.ds(r, CHD)], wbuf[b], semsc[b]).wait()
        pltpu.async_copy(wbuf[b], deg_out.at[c, pl.ds(r, CHD)], semw[b])
    for k in range(NWOD - 2, NWOD):
        b = k % 2
        pltpu.make_async_copy(
            wbuf[b], deg_out.at[c, pl.ds(row0 + k * CHD, CHD)], semw[b]
        ).wait()


def _tc_body(p_ref, g_ref, f_ref, w_ref, b_ref, o_ref):
    ssum = p_ref[0] + p_ref[1]
    deg = g_ref[0, :, 0:1] + g_ref[1, :, 0:1]
    agg = jnp.where(deg > 0.0, ssum / jnp.maximum(deg, 1.0), f_ref[...])
    h = lax.dot_general(agg, w_ref[...], (((1,), (1,)), ((), ())),
                        preferred_element_type=jnp.float32)
    o_ref[...] = jnp.maximum(h + b_ref[...], 0.0)


@jax.jit
def kernel(feature, edge_index, W, b):
    src = edge_index[0].astype(jnp.int32)
    dst = edge_index[1].astype(jnp.int32)
    ppw = EPW - N_EDGES // NW
    dummy_dst = jnp.broadcast_to(N_NODES + jnp.arange(ppw, dtype=jnp.int32),
                                 (NW, ppw))
    src = jnp.concatenate(
        [src.reshape(NW, -1), jnp.zeros((NW, ppw), jnp.int32)], axis=1)
    dst = jnp.concatenate([dst.reshape(NW, -1), dummy_dst], axis=1)
    src = src.reshape(-1)
    dst = dst.reshape(-1)
    zfeat = jnp.zeros((CH, D_IN), jnp.float32)
    zdeg = jnp.zeros((CHD, DEG_W), jnp.float32)
    ones = jnp.ones((CHD, DEG_W), jnp.float32)

    mesh = plsc.VectorSubcoreMesh(core_axis_name="c", subcore_axis_name="s")
    acc_call = pl.kernel(
        _acc_body,
        out_type=jax.ShapeDtypeStruct((NC, N_PAD, D_IN), jnp.float32),
        mesh=mesh,
        scratch_types=[
            pltpu.VMEM((CH,), jnp.int32),
            pltpu.VMEM((CH,), jnp.int32),
            pltpu.VMEM((CH, D_IN), jnp.float32),
            pltpu.SemaphoreType.DMA,
            pltpu.VMEM_SHARED((N_PAD, D_IN), jnp.float32),
        ],
    )
    partial = acc_call(feature, src, dst, zfeat)

    deg_call = pl.kernel(
        _deg_body,
        out_type=jax.ShapeDtypeStruct((NC, N_PAD, DEG_W), jnp.float32),
        mesh=mesh,
        scratch_types=[
            tuple(pltpu.VMEM((CHD,), jnp.int32) for _ in range(NBUF_D)),
            pltpu.VMEM((CHD, DEG_W), jnp.float32),
            tuple(pltpu.VMEM((CHD, DEG_W), jnp.float32) for _ in range(2)),
            tuple(pltpu.SemaphoreType.DMA for _ in range(NBUF_D)),
            tuple(pltpu.SemaphoreType.DMA for _ in range(NBUF_D)),
            tuple(pltpu.SemaphoreType.DMA for _ in range(2)),
            pltpu.VMEM_SHARED((N_PAD, DEG_W), jnp.float32),
        ],
    )
    pdeg = deg_call(dst, zdeg, ones)

    R = 1000
    out = pl.pallas_call(
        _tc_body,
        grid=(N_NODES // R,),
        in_specs=[
            pl.BlockSpec((NC, R, D_IN), lambda i: (0, i, 0)),
            pl.BlockSpec((NC, R, DEG_W), lambda i: (0, i, 0)),
            pl.BlockSpec((R, D_IN), lambda i: (i, 0)),
            pl.BlockSpec((D_OUT, D_IN), lambda i: (0, 0)),
            pl.BlockSpec((1, D_OUT), lambda i: (0, 0)),
        ],
        out_specs=pl.BlockSpec((R, D_OUT), lambda i: (i, 0)),
        out_shape=jax.ShapeDtypeStruct((N_NODES, D_OUT), jnp.float32),
    )(partial, pdeg, feature, W, b.reshape(1, D_OUT))
    return out


# speedup vs baseline: 1.6402x; 1.6402x over previous
import jax
import jax.numpy as jnp
from jax import lax
from jax.experimental import pallas as pl
from jax.experimental.pallas import tpu as pltpu
from jax.experimental.pallas import tpu_sc as plsc

N_NODES = 10000
N_EDGES = 320000
D_IN = 128
D_OUT = 128

NC = 2
NS = 16
NW = NC * NS
EPW = 10240
PAD_E = NW * EPW - N_EDGES
EPW_A = N_EDGES // NW
CH = 80
NCH = EPW_A // CH
CHD = 64
NCHD = EPW // CHD
NBUF_D = 5
N_PAD = 10240
RPT = N_PAD // NS
NWO = RPT // CH
NWOD = RPT // CHD
DEG_W = 128


def _acc_body(feat_hbm, src_hbm, dst_hbm, zfeat_hbm,
              acc_out,
              idx_s, idx_d, rows, sem, acc_sh):
    c = lax.axis_index("c")
    s = lax.axis_index("s")
    wid = s * NC + c
    base = wid * EPW_A
    row0 = s * RPT

    pltpu.sync_copy(zfeat_hbm, rows)

    def zinit(k, carry):
        pltpu.sync_copy(rows, acc_sh.at[pl.ds(row0 + k * CH, CH)])
        return carry

    lax.fori_loop(0, NWO, zinit, 0)
    plsc.subcore_barrier()

    def chunk(j, carry):
        off = base + j * CH
        pltpu.sync_copy(src_hbm.at[pl.ds(off, CH)], idx_s)
        pltpu.sync_copy(dst_hbm.at[pl.ds(off, CH)], idx_d)
        pltpu.async_copy(feat_hbm.at[idx_s], rows, sem).wait()
        pltpu.sync_copy(rows, acc_sh.at[idx_d], add=True)
        return carry

    lax.fori_loop(0, NCH, chunk, 0)
    plsc.subcore_barrier()

    def wout(k, carry):
        r = row0 + k * CH
        pltpu.sync_copy(acc_sh.at[pl.ds(r, CH)], rows)
        pltpu.sync_copy(rows, acc_out.at[c, pl.ds(r, CH)])
        return carry

    lax.fori_loop(0, NWO, wout, 0)


def _deg_body(dst_hbm, zdeg_hbm, ones_hbm,
              deg_out,
              idx_d, ones_v, wbuf, semid, semsc, semw, deg_sh):
    c = lax.axis_index("c")
    s = lax.axis_index("s")
    wid = s * NC + c
    base = wid * EPW
    row0 = s * RPT

    def idx_start(j, b):
        pltpu.async_copy(dst_hbm.at[pl.ds(base + j * CHD, CHD)], idx_d[b], semid[b])

    def idx_wait(j, b):
        pltpu.make_async_copy(
            dst_hbm.at[pl.ds(base + j * CHD, CHD)], idx_d[b], semid[b]
        ).wait()

    def scatter_start(b):
        pltpu.async_copy(ones_v, deg_sh.at[idx_d[b]], semsc[b], add=True)

    def scatter_wait(b):
        pltpu.make_async_copy(ones_v, deg_sh.at[idx_d[b]], semsc[b]).wait()

    pltpu.sync_copy(ones_hbm, ones_v)
    pltpu.sync_copy(zdeg_hbm, wbuf[0])

    def zinit(k, carry):
        pltpu.sync_copy(wbuf[0], deg_sh.at[pl.ds(row0 + k * CHD, CHD)])
        return carry

    lax.fori_loop(0, NWOD, zinit, 0)
    plsc.subcore_barrier()

    idx_start(0, 0)
    idx_start(1, 1)

    def grp(g, carry):
        for b in range(NBUF_D):
            j = g * NBUF_D + b
            bi = (b + 2) % NBUF_D

            @pl.when(j >= 3)
            def _():
                scatter_wait(bi)

            @pl.when(j + 2 < NCHD)
            def _():
                idx_start(j + 2, bi)

            idx_wait(j, b)
            scatter_start(b)
        return carry

    lax.fori_loop(0, NCHD // NBUF_D, grp, 0)
    for j0 in range(NCHD - 3, NCHD):
        scatter_wait(j0 % NBUF_D)
    plsc.subcore_barrier()

    for k in range(NWOD):
        b = k % 2
        if k >= 2:
            pltpu.make_async_copy(
                wbuf[b], deg_out.at[c, pl.ds(row0 + (k - 2) * CHD, CHD)], semw[b]
            ).wait()
        r = row0 + k * CHD
        pltpu.async_copy(deg_sh.at[pl.ds(r, CHD)], wbuf[b], semsc[b]).wait()
        pltpu.async_copy(wbuf[b], deg_out.at[c, pl.ds(r, CHD)], semw[b])
    for k in range(NWOD - 2, NWOD):
        b = k % 2
        pltpu.make_async_copy(
            wbuf[b], deg_out.at[c, pl.ds(row0 + k * CHD, CHD)], semw[b]
        ).wait()


def _tc_body(p_ref, g_ref, f_ref, w_ref, b_ref, o_ref):
    ssum = p_ref[0] + p_ref[1]
    deg = g_ref[0, :, 0:1] + g_ref[1, :, 0:1]
    agg = jnp.where(deg > 0.0, ssum / jnp.maximum(deg, 1.0), f_ref[...])
    h = lax.dot_general(agg, w_ref[...], (((1,), (1,)), ((), ())),
                        preferred_element_type=jnp.float32)
    o_ref[...] = jnp.maximum(h + b_ref[...], 0.0)


@jax.jit
def kernel(feature, edge_index, W, b):
    src = edge_index[0].astype(jnp.int32)
    dst = edge_index[1].astype(jnp.int32)
    ppw = EPW - EPW_A
    dummy_dst = jnp.broadcast_to(N_NODES + jnp.arange(ppw, dtype=jnp.int32),
                                 (NW, ppw))
    dst_p = jnp.concatenate([dst.reshape(NW, -1), dummy_dst],
                            axis=1).reshape(-1)
    zfeat = jnp.zeros((CH, D_IN), jnp.float32)
    zdeg = jnp.zeros((CHD, DEG_W), jnp.float32)
    ones = jnp.ones((CHD, DEG_W), jnp.float32)

    mesh = plsc.VectorSubcoreMesh(core_axis_name="c", subcore_axis_name="s")
    acc_call = pl.kernel(
        _acc_body,
        out_type=jax.ShapeDtypeStruct((NC, N_PAD, D_IN), jnp.float32),
        mesh=mesh,
        scratch_types=[
            pltpu.VMEM((CH,), jnp.int32),
            pltpu.VMEM((CH,), jnp.int32),
            pltpu.VMEM((CH, D_IN), jnp.float32),
            pltpu.SemaphoreType.DMA,
            pltpu.VMEM_SHARED((N_PAD, D_IN), jnp.float32),
        ],
    )
    partial = acc_call(feature, src, dst, zfeat)

    deg_call = pl.kernel(
        _deg_body,
        out_type=jax.ShapeDtypeStruct((NC, N_PAD, DEG_W), jnp.float32),
        mesh=mesh,
        scratch_types=[
            tuple(pltpu.VMEM((CHD,), jnp.int32) for _ in range(NBUF_D)),
            pltpu.VMEM((CHD, DEG_W), jnp.float32),
            tuple(pltpu.VMEM((CHD, DEG_W), jnp.float32) for _ in range(2)),
            tuple(pltpu.SemaphoreType.DMA for _ in range(NBUF_D)),
            tuple(pltpu.SemaphoreType.DMA for _ in range(NBUF_D)),
            tuple(pltpu.SemaphoreType.DMA for _ in range(2)),
            pltpu.VMEM_SHARED((N_PAD, DEG_W), jnp.float32),
        ],
    )
    pdeg = deg_call(dst_p, zdeg, ones)

    R = 1000
    out = pl.pallas_call(
        _tc_body,
        grid=(N_NODES // R,),
        in_specs=[
            pl.BlockSpec((NC, R, D_IN), lambda i: (0, i, 0)),
            pl.BlockSpec((NC, R, DEG_W), lambda i: (0, i, 0)),
            pl.BlockSpec((R, D_IN), lambda i: (i, 0)),
            pl.BlockSpec((D_OUT, D_IN), lambda i: (0, 0)),
            pl.BlockSpec((1, D_OUT), lambda i: (0, 0)),
        ],
        out_specs=pl.BlockSpec((R, D_OUT), lambda i: (i, 0)),
        out_shape=jax.ShapeDtypeStruct((N_NODES, D_OUT), jnp.float32),
    )(partial, pdeg, feature, W, b.reshape(1, D_OUT))
    return out
